# baseline (speedup 1.0000x reference)
import jax
import jax.numpy as jnp
from jax.experimental import pallas as pl
from jax.experimental.pallas import tpu as pltpu

_B, _S, _VOCAB = 1024, 50, 1000
_BB = 16
_NST = _B // _BB
_NBUF = 8


def _body(idx_ref, o_ref, *scr):
    bufs, sems = scr[:_NBUF], scr[_NBUF:]
    for g in range(_NST):
        b = g % _NBUF
        if g >= _NBUF:
            pltpu.make_async_copy(
                bufs[b], o_ref.at[pl.ds((g - _NBUF) * _BB, _BB)],
                sems[b]).wait()
        idx = idx_ref[pl.ds(g * _BB, _BB), :]
        cols = jax.lax.broadcasted_iota(jnp.int32, (_BB, _S, _VOCAB), 2)
        bufs[b][...] = (cols == idx[:, :, None]).astype(jnp.float32)
        pltpu.async_copy(bufs[b], o_ref.at[pl.ds(g * _BB, _BB)], sems[b])
    for b in range(_NBUF):
        g = _NST - _NBUF + b
        pltpu.make_async_copy(
            bufs[b], o_ref.at[pl.ds(g * _BB, _BB)], sems[b]).wait()


_tc_onehot = pl.pallas_call(
    _body,
    out_shape=jax.ShapeDtypeStruct((_B, _S, _VOCAB), jnp.float32),
    in_specs=[pl.BlockSpec(memory_space=pltpu.MemorySpace.VMEM)],
    out_specs=pl.BlockSpec(memory_space=pltpu.MemorySpace.HBM),
    scratch_shapes=(
        [pltpu.VMEM((_BB, _S, _VOCAB), jnp.float32)] * _NBUF
        + [pltpu.SemaphoreType.DMA] * _NBUF
    ),
)


def kernel(inputs):
    return _tc_onehot(inputs.astype(jnp.int32))


# speedup vs baseline: 1.2197x; 1.2197x over previous
import jax
import jax.numpy as jnp
from jax.experimental import pallas as pl

_B, _S, _VOCAB = 1024, 50, 1000
_SP, _VP = 56, 1024
_BB = 64


def _body(idx_ref, o_ref):
    idx = idx_ref[...]
    cols = jax.lax.broadcasted_iota(jnp.int32, (_BB, _SP, _VP), 2)
    o_ref[...] = (cols == idx[:, :, None]).astype(jnp.float32)


_tc_onehot = pl.pallas_call(
    _body,
    out_shape=jax.ShapeDtypeStruct((_B, _SP, _VP), jnp.float32),
    grid=(_B // _BB,),
    in_specs=[pl.BlockSpec((_BB, _SP), lambda i: (i, 0))],
    out_specs=pl.BlockSpec((_BB, _SP, _VP), lambda i: (i, 0, 0)),
)


def kernel(inputs):
    idx = inputs.astype(jnp.int32)
    idx = jnp.pad(idx, ((0, 0), (0, _SP - _S)), constant_values=-1)
    return _tc_onehot(idx)[:, :_S, :_VOCAB]
